# baseline (speedup 1.0000x reference)
import jax
import jax.numpy as jnp
from jax.experimental import pallas as pl
from jax.experimental.pallas import tpu as pltpu

N, F_IN, F_OUT = 10000, 128, 16
TILE_ROWS = 200
NB = 4
STEPS = N // TILE_ROWS


def _gcn_kernel(x_ref, adj_hbm, w_ref, b_ref, out_ref, bufs, support_ref, sems):
    def dma(i, slot):
        return pltpu.make_async_copy(
            adj_hbm.at[pl.ds(i * TILE_ROWS, TILE_ROWS), :],
            bufs.at[slot],
            sems.at[slot],
        )

    for i in range(NB - 1):
        dma(i, i).start()

    support_ref[...] = jnp.dot(
        x_ref[...], w_ref[...], preferred_element_type=jnp.float32
    )

    for i in range(STEPS):
        slot = i % NB
        dma(i, slot).wait()
        out_ref[pl.ds(i * TILE_ROWS, TILE_ROWS), :] = (
            jnp.dot(
                bufs[slot], support_ref[...], preferred_element_type=jnp.float32
            )
            + b_ref[...]
        )
        nxt = i + NB - 1
        if nxt < STEPS:
            dma(nxt, nxt % NB).start()


@jax.jit
def kernel(x, adj, W, b):
    b2 = b.reshape(1, F_OUT)
    return pl.pallas_call(
        _gcn_kernel,
        in_specs=[
            pl.BlockSpec((N, F_IN), lambda: (0, 0)),
            pl.BlockSpec(memory_space=pltpu.HBM),
            pl.BlockSpec((F_IN, F_OUT), lambda: (0, 0)),
            pl.BlockSpec((1, F_OUT), lambda: (0, 0)),
        ],
        out_specs=pl.BlockSpec((N, F_OUT), lambda: (0, 0)),
        out_shape=jax.ShapeDtypeStruct((N, F_OUT), jnp.float32),
        scratch_shapes=[
            pltpu.VMEM((NB, TILE_ROWS, N), jnp.float32),
            pltpu.VMEM((N, F_OUT), jnp.float32),
            pltpu.SemaphoreType.DMA((NB,)),
        ],
    )(x, adj, W, b2)
